# baseline (speedup 1.0000x reference)
import functools

import jax
import jax.numpy as jnp
from jax import lax
from jax.experimental import pallas as pl
from jax.experimental.pallas import tpu as pltpu

N = 10000
D = 128
NCL = 16
NH = 3 * NCL
NP = 10240
BR = 512
GRID = NP // BR

E = 320000
NC = 2
NS = 16
W = NC * NS
CH = 128
CPW = 80
EPAD = W * CPW * CH


def _sigmoid(x):
    return 1.0 / (1.0 + jnp.exp(-x))


def _layer_body(acc_ref, deg_ref, x_ref, wl_ref, bl_ref, wr_ref, o_ref, *, act):
    a = jnp.sum(acc_ref[...], axis=0)
    deg = jnp.sum(deg_ref[...], axis=0)[:, 0]
    mean = a / jnp.maximum(deg, 1.0)[:, None]
    out = (jnp.dot(mean, wl_ref[...], preferred_element_type=jnp.float32)
           + bl_ref[...]
           + jnp.dot(x_ref[...], wr_ref[...], preferred_element_type=jnp.float32))
    if act == "relu":
        out = jnp.maximum(out, 0.0)
    else:
        out = _sigmoid(out)
    o_ref[...] = out


def _layer1_body(acc_ref, degf_ref, x_ref, wl_ref, bl_ref, wr_ref,
                 o_ref, degc_ref):
    a = jnp.sum(acc_ref[...], axis=0)
    deg = degf_ref[0, :, 0] + degf_ref[1, :, 0]
    degc_ref[...] = jnp.broadcast_to(deg[:, None], (BR, NCL))
    mean = a / jnp.maximum(deg, 1.0)[:, None]
    out = (jnp.dot(mean, wl_ref[...], preferred_element_type=jnp.float32)
           + bl_ref[...]
           + jnp.dot(x_ref[...], wr_ref[...], preferred_element_type=jnp.float32))
    o_ref[...] = jnp.maximum(out, 0.0)


def _layer1_call(acc2, degf, x, wl, bl, wr):
    A = acc2.shape[0]
    return pl.pallas_call(
        _layer1_body,
        grid=(GRID,),
        in_specs=[
            pl.BlockSpec((A, BR, D), lambda i: (0, i, 0)),
            pl.BlockSpec((NC, BR, D), lambda i: (0, i, 0)),
            pl.BlockSpec((BR, D), lambda i: (i, 0)),
            pl.BlockSpec((D, D), lambda i: (0, 0)),
            pl.BlockSpec((1, D), lambda i: (0, 0)),
            pl.BlockSpec((D, D), lambda i: (0, 0)),
        ],
        out_specs=[
            pl.BlockSpec((BR, D), lambda i: (i, 0)),
            pl.BlockSpec((BR, NCL), lambda i: (i, 0)),
        ],
        out_shape=[
            jax.ShapeDtypeStruct((NP, D), jnp.float32),
            jax.ShapeDtypeStruct((NP, NCL), jnp.float32),
        ],
    )(acc2, degf, x, wl, bl.reshape(1, -1), wr)


def _loss_body(acch_ref, deg_ref, h_ref, wlcat_ref, wrcat_ref, blcat_ref,
               y_ref, o_ref):
    i = pl.program_id(0)
    a = jnp.sum(acch_ref[...], axis=0)
    deg = jnp.sum(deg_ref[...], axis=0)[:, 0]
    mean_h = a / jnp.maximum(deg, 1.0)[:, None]
    o = _sigmoid(jnp.dot(mean_h, wlcat_ref[...], preferred_element_type=jnp.float32)
                 + blcat_ref[...]
                 + jnp.dot(h_ref[...], wrcat_ref[...], preferred_element_type=jnp.float32))
    rows = i * BR + lax.broadcasted_iota(jnp.int32, (BR, 1), 0)
    valid = rows < N
    cls = lax.broadcasted_iota(jnp.int32, (BR, NCL), 1)
    total = jnp.float32(0.0)
    for k in range(3):
        lg = o[:, k * NCL:(k + 1) * NCL]
        m = jnp.max(lg, axis=1, keepdims=True)
        lse = jnp.log(jnp.sum(jnp.exp(lg - m), axis=1, keepdims=True)) + m
        logp = lg - lse
        sel = jnp.sum(jnp.where(cls == y_ref[:, k:k + 1], logp, 0.0),
                      axis=1, keepdims=True)
        total = total + jnp.sum(jnp.where(valid, sel, 0.0))

    @pl.when(i == 0)
    def _():
        o_ref[...] = jnp.zeros_like(o_ref)

    o_ref[...] += jnp.reshape(-total / N, (1, 1))


def _layer_call(acc2, deg2, x, wl, bl, wr, act):
    A = acc2.shape[0]
    dout = wl.shape[1]
    return pl.pallas_call(
        functools.partial(_layer_body, act=act),
        grid=(GRID,),
        in_specs=[
            pl.BlockSpec((A, BR, D), lambda i: (0, i, 0)),
            pl.BlockSpec((A, BR, NCL), lambda i: (0, i, 0)),
            pl.BlockSpec((BR, D), lambda i: (i, 0)),
            pl.BlockSpec((D, dout), lambda i: (0, 0)),
            pl.BlockSpec((1, dout), lambda i: (0, 0)),
            pl.BlockSpec((D, dout), lambda i: (0, 0)),
        ],
        out_specs=pl.BlockSpec((BR, dout), lambda i: (i, 0)),
        out_shape=jax.ShapeDtypeStruct((NP, dout), jnp.float32),
    )(acc2, deg2, x, wl, bl.reshape(1, -1), wr)


def _loss_call(acch2, deg2, h, wlcat, wrcat, blcat, y):
    A = acch2.shape[0]
    return pl.pallas_call(
        _loss_body,
        grid=(GRID,),
        in_specs=[
            pl.BlockSpec((A, BR, D), lambda i: (0, i, 0)),
            pl.BlockSpec((A, BR, NCL), lambda i: (0, i, 0)),
            pl.BlockSpec((BR, D), lambda i: (i, 0)),
            pl.BlockSpec((D, NH), lambda i: (0, 0)),
            pl.BlockSpec((D, NH), lambda i: (0, 0)),
            pl.BlockSpec((1, NH), lambda i: (0, 0)),
            pl.BlockSpec((BR, 3), lambda i: (i, 0)),
        ],
        out_specs=pl.BlockSpec((1, 1), lambda i: (0, 0)),
        out_shape=jax.ShapeDtypeStruct((1, 1), jnp.float32),
    )(acch2, deg2, h, wlcat, wrcat, blcat.reshape(1, -1), y)


from jax.experimental.pallas import tpu_sc as plsc

RT = NP // NS


def _sc_mesh():
    return plsc.VectorSubcoreMesh(core_axis_name="c", subcore_axis_name="s",
                                  num_cores=NC, num_subcores=NS)


NCH = EPAD // CH


def _worker_start(c, s):
    wid = s * NC + c
    return wid * CPW


def _sc_agg_call(x_pad, src_f, dst_f, d):
    scratch = [
        pltpu.VMEM((CPW, CH), jnp.int32),
        pltpu.VMEM((CPW, CH), jnp.int32),
        pltpu.VMEM((CH, d), jnp.float32),
        pltpu.VMEM_SHARED((NP, d), jnp.float32),
        pltpu.SemaphoreType.DMA,
    ]

    def body(x_hbm, src_hbm, dst_hbm, zrows_hbm, out_hbm,
             src_v, dst_v, rows_v, acc, sem):
        c = lax.axis_index("c")
        s = lax.axis_index("s")
        base = s * RT
        start = _worker_start(c, s)
        pltpu.sync_copy(zrows_hbm.at[pl.ds(base, RT)], acc.at[pl.ds(base, RT)])
        pltpu.sync_copy(src_hbm.at[pl.ds(start, CPW)], src_v)
        pltpu.sync_copy(dst_hbm.at[pl.ds(start, CPW)], dst_v)
        plsc.subcore_barrier()

        @pl.loop(0, CPW)
        def _chunk(j):
            pltpu.async_copy(x_hbm.at[src_v.at[j]], rows_v, sem).wait()
            pltpu.sync_copy(rows_v, acc.at[dst_v.at[j]], add=True)

        plsc.subcore_barrier()
        pltpu.sync_copy(acc.at[pl.ds(base, RT)], out_hbm.at[c, pl.ds(base, RT)])

    fn = pl.kernel(body,
                   out_type=jax.ShapeDtypeStruct((NC, NP, d), jnp.float32),
                   mesh=_sc_mesh(), scratch_types=scratch)
    return fn(x_pad, src_f, dst_f, jnp.zeros((NP, d), jnp.float32))


def _sc_deg_call(dst_f):
    scratch = [
        pltpu.VMEM((CPW, CH), jnp.int32),
        pltpu.VMEM((CH, D), jnp.float32),
        pltpu.VMEM_SHARED((NP, D), jnp.float32),
    ]

    def body(dst_hbm, zdeg_hbm, ones_hbm, degout_hbm, dst_v, ones_v, dacc):
        c = lax.axis_index("c")
        s = lax.axis_index("s")
        base = s * RT
        start = _worker_start(c, s)
        pltpu.sync_copy(zdeg_hbm.at[pl.ds(base, RT)], dacc.at[pl.ds(base, RT)])
        pltpu.sync_copy(ones_hbm, ones_v)
        pltpu.sync_copy(dst_hbm.at[pl.ds(start, CPW)], dst_v)
        plsc.subcore_barrier()

        @pl.loop(0, CPW)
        def _chunk(j):
            pltpu.sync_copy(ones_v, dacc.at[dst_v.at[j]], add=True)

        plsc.subcore_barrier()
        pltpu.sync_copy(dacc.at[pl.ds(base, RT)],
                        degout_hbm.at[c, pl.ds(base, RT)])

    fn = pl.kernel(body,
                   out_type=jax.ShapeDtypeStruct((NC, NP, D), jnp.float32),
                   mesh=_sc_mesh(), scratch_types=scratch)
    return fn(dst_f, jnp.zeros((NP, D), jnp.float32),
              jnp.ones((CH, D), jnp.float32))


def kernel(X, adj, y, enc0_Wl, enc0_bl, enc0_Wr, enc1_Wl, enc1_bl, enc1_Wr,
           enc2_Wl, enc2_bl, enc2_Wr, mt_Wl, mt_bl, mt_Wr, mt2_Wl, mt2_bl,
           mt2_Wr, mt3_Wl, mt3_bl, mt3_Wr):
    x = X[0, 0]
    x_pad = jnp.pad(x, ((0, NP - N), (0, 0)))
    src, dst = adj[0], adj[1]
    y_pad = jnp.pad(y[0], ((0, NP - N), (0, 0)))

    wl_cat = jnp.concatenate([mt_Wl, mt2_Wl, mt3_Wl], axis=1)
    wr_cat = jnp.concatenate([mt_Wr, mt2_Wr, mt3_Wr], axis=1)
    bl_cat = jnp.concatenate([mt_bl, mt2_bl, mt3_bl], axis=0)

    src_w = jnp.concatenate([src, jnp.zeros((EPAD - E,), jnp.int32)]
                            ).reshape(NCH, CH)
    dst_w = jnp.concatenate([dst, jnp.full((EPAD - E,), N, jnp.int32)]
                            ).reshape(NCH, CH)

    degf = _sc_deg_call(dst_w)
    acc0 = _sc_agg_call(x_pad, src_w, dst_w, D)
    h1, degc = _layer1_call(acc0, degf, x_pad, enc0_Wl, enc0_bl, enc0_Wr)
    deg2 = degc[None]
    acc1 = _sc_agg_call(h1, src_w, dst_w, D)
    h2 = _layer_call(acc1, deg2, h1, enc1_Wl, enc1_bl, enc1_Wr, "relu")
    acc2 = _sc_agg_call(h2, src_w, dst_w, D)
    h3 = _layer_call(acc2, deg2, h2, enc2_Wl, enc2_bl, enc2_Wr, "sigmoid")
    acch = _sc_agg_call(h3, src_w, dst_w, D)
    loss = _loss_call(acch, deg2, h3, wl_cat, wr_cat, bl_cat, y_pad)
    return loss[0, 0]


# speedup vs baseline: 1.4856x; 1.4856x over previous
import functools

import jax
import jax.numpy as jnp
from jax import lax
from jax.experimental import pallas as pl
from jax.experimental.pallas import tpu as pltpu

N = 10000
D = 128
NCL = 16
NH = 3 * NCL
NP = 10240
BR = 512
GRID = NP // BR

E = 320000
NC = 2
NS = 16
W = NC * NS
CH = 128
CPW = 79
EPAD = W * CPW * CH


def _sigmoid(x):
    return 1.0 / (1.0 + jnp.exp(-x))


def _layer_body(acc_ref, deg_ref, x_ref, wl_ref, bl_ref, wr_ref, o_ref, *, act):
    a = jnp.sum(acc_ref[...], axis=0)
    deg = jnp.sum(deg_ref[...], axis=0)[:, 0]
    mean = a / jnp.maximum(deg, 1.0)[:, None]
    out = (jnp.dot(mean, wl_ref[...], preferred_element_type=jnp.float32)
           + bl_ref[...]
           + jnp.dot(x_ref[...], wr_ref[...], preferred_element_type=jnp.float32))
    if act == "relu":
        out = jnp.maximum(out, 0.0)
    else:
        out = _sigmoid(out)
    o_ref[...] = out


def _layer1_body(acc_ref, degf_ref, x_ref, wl_ref, bl_ref, wr_ref,
                 o_ref, degc_ref):
    a = jnp.sum(acc_ref[...], axis=0)
    deg = degf_ref[0, :, 0] + degf_ref[1, :, 0]
    degc_ref[...] = jnp.broadcast_to(deg[:, None], (BR, NCL))
    mean = a / jnp.maximum(deg, 1.0)[:, None]
    out = (jnp.dot(mean, wl_ref[...], preferred_element_type=jnp.float32)
           + bl_ref[...]
           + jnp.dot(x_ref[...], wr_ref[...], preferred_element_type=jnp.float32))
    o_ref[...] = jnp.maximum(out, 0.0)


def _layer1_call(acc2, degf, x, wl, bl, wr):
    A = acc2.shape[0]
    return pl.pallas_call(
        _layer1_body,
        grid=(GRID,),
        in_specs=[
            pl.BlockSpec((A, BR, D), lambda i: (0, i, 0)),
            pl.BlockSpec((NC, BR, D), lambda i: (0, i, 0)),
            pl.BlockSpec((BR, D), lambda i: (i, 0)),
            pl.BlockSpec((D, D), lambda i: (0, 0)),
            pl.BlockSpec((1, D), lambda i: (0, 0)),
            pl.BlockSpec((D, D), lambda i: (0, 0)),
        ],
        out_specs=[
            pl.BlockSpec((BR, D), lambda i: (i, 0)),
            pl.BlockSpec((BR, NCL), lambda i: (i, 0)),
        ],
        out_shape=[
            jax.ShapeDtypeStruct((NP, D), jnp.float32),
            jax.ShapeDtypeStruct((NP, NCL), jnp.float32),
        ],
    )(acc2, degf, x, wl, bl.reshape(1, -1), wr)


def _loss_body(acch_ref, deg_ref, h_ref, wlcat_ref, wrcat_ref, blcat_ref,
               y_ref, o_ref):
    i = pl.program_id(0)
    a = jnp.sum(acch_ref[...], axis=0)
    deg = jnp.sum(deg_ref[...], axis=0)[:, 0]
    mean_h = a / jnp.maximum(deg, 1.0)[:, None]
    o = _sigmoid(jnp.dot(mean_h, wlcat_ref[...], preferred_element_type=jnp.float32)
                 + blcat_ref[...]
                 + jnp.dot(h_ref[...], wrcat_ref[...], preferred_element_type=jnp.float32))
    rows = i * BR + lax.broadcasted_iota(jnp.int32, (BR, 1), 0)
    valid = rows < N
    cls = lax.broadcasted_iota(jnp.int32, (BR, NCL), 1)
    total = jnp.float32(0.0)
    for k in range(3):
        lg = o[:, k * NCL:(k + 1) * NCL]
        m = jnp.max(lg, axis=1, keepdims=True)
        lse = jnp.log(jnp.sum(jnp.exp(lg - m), axis=1, keepdims=True)) + m
        logp = lg - lse
        sel = jnp.sum(jnp.where(cls == y_ref[:, k:k + 1], logp, 0.0),
                      axis=1, keepdims=True)
        total = total + jnp.sum(jnp.where(valid, sel, 0.0))

    @pl.when(i == 0)
    def _():
        o_ref[...] = jnp.zeros_like(o_ref)

    o_ref[...] += jnp.reshape(-total / N, (1, 1))


def _layer_call(acc2, deg2, x, wl, bl, wr, act):
    A = acc2.shape[0]
    dout = wl.shape[1]
    return pl.pallas_call(
        functools.partial(_layer_body, act=act),
        grid=(GRID,),
        in_specs=[
            pl.BlockSpec((A, BR, D), lambda i: (0, i, 0)),
            pl.BlockSpec((A, BR, NCL), lambda i: (0, i, 0)),
            pl.BlockSpec((BR, D), lambda i: (i, 0)),
            pl.BlockSpec((D, dout), lambda i: (0, 0)),
            pl.BlockSpec((1, dout), lambda i: (0, 0)),
            pl.BlockSpec((D, dout), lambda i: (0, 0)),
        ],
        out_specs=pl.BlockSpec((BR, dout), lambda i: (i, 0)),
        out_shape=jax.ShapeDtypeStruct((NP, dout), jnp.float32),
    )(acc2, deg2, x, wl, bl.reshape(1, -1), wr)


def _loss_call(acch2, deg2, h, wlcat, wrcat, blcat, y):
    A = acch2.shape[0]
    return pl.pallas_call(
        _loss_body,
        grid=(GRID,),
        in_specs=[
            pl.BlockSpec((A, BR, D), lambda i: (0, i, 0)),
            pl.BlockSpec((A, BR, NCL), lambda i: (0, i, 0)),
            pl.BlockSpec((BR, D), lambda i: (i, 0)),
            pl.BlockSpec((D, NH), lambda i: (0, 0)),
            pl.BlockSpec((D, NH), lambda i: (0, 0)),
            pl.BlockSpec((1, NH), lambda i: (0, 0)),
            pl.BlockSpec((BR, 3), lambda i: (i, 0)),
        ],
        out_specs=pl.BlockSpec((1, 1), lambda i: (0, 0)),
        out_shape=jax.ShapeDtypeStruct((1, 1), jnp.float32),
    )(acch2, deg2, h, wlcat, wrcat, blcat.reshape(1, -1), y)


from jax.experimental.pallas import tpu_sc as plsc

RT = NP // NS


def _sc_mesh():
    return plsc.VectorSubcoreMesh(core_axis_name="c", subcore_axis_name="s",
                                  num_cores=NC, num_subcores=NS)


NCH = EPAD // CH


def _worker_start(c, s):
    wid = s * NC + c
    return wid * CPW


def _sc_agg_call(x_pad, src_f, dst_f, d):
    scratch = [
        pltpu.VMEM((CPW, CH), jnp.int32),
        pltpu.VMEM((CPW, CH), jnp.int32),
        pltpu.VMEM((CH, d), jnp.float32),
        pltpu.VMEM_SHARED((NP, d), jnp.float32),
        pltpu.SemaphoreType.DMA,
    ]

    def body(x_hbm, src_hbm, dst_hbm, zrows_hbm, out_hbm,
             src_v, dst_v, rows_v, acc, sem):
        c = lax.axis_index("c")
        s = lax.axis_index("s")
        base = s * RT
        wid = s * NC + c
        pltpu.sync_copy(zrows_hbm.at[pl.ds(base, RT)], acc.at[pl.ds(base, RT)])
        pltpu.sync_copy(src_hbm.at[wid], src_v)
        pltpu.sync_copy(dst_hbm.at[wid], dst_v)
        plsc.subcore_barrier()

        @pl.loop(0, CPW)
        def _chunk(j):
            pltpu.async_copy(x_hbm.at[src_v.at[j]], rows_v, sem).wait()
            pltpu.sync_copy(rows_v, acc.at[dst_v.at[j]], add=True)

        plsc.subcore_barrier()
        pltpu.sync_copy(acc.at[pl.ds(base, RT)], out_hbm.at[c, pl.ds(base, RT)])

    fn = pl.kernel(body,
                   out_type=jax.ShapeDtypeStruct((NC, NP, d), jnp.float32),
                   mesh=_sc_mesh(), scratch_types=scratch)
    return fn(x_pad, src_f, dst_f, jnp.zeros((NP, d), jnp.float32))


def _sc_deg_call(dst_f):
    scratch = [
        pltpu.VMEM((CPW, CH), jnp.int32),
        pltpu.VMEM((CH, D), jnp.float32),
        pltpu.VMEM_SHARED((NP, D), jnp.float32),
    ]

    def body(dst_hbm, zdeg_hbm, ones_hbm, degout_hbm, dst_v, ones_v, dacc):
        c = lax.axis_index("c")
        s = lax.axis_index("s")
        base = s * RT
        wid = s * NC + c
        pltpu.sync_copy(zdeg_hbm.at[pl.ds(base, RT)], dacc.at[pl.ds(base, RT)])
        pltpu.sync_copy(ones_hbm, ones_v)
        pltpu.sync_copy(dst_hbm.at[wid], dst_v)
        plsc.subcore_barrier()

        @pl.loop(0, CPW)
        def _chunk(j):
            pltpu.sync_copy(ones_v, dacc.at[dst_v.at[j]], add=True)

        plsc.subcore_barrier()
        pltpu.sync_copy(dacc.at[pl.ds(base, RT)],
                        degout_hbm.at[c, pl.ds(base, RT)])

    fn = pl.kernel(body,
                   out_type=jax.ShapeDtypeStruct((NC, NP, D), jnp.float32),
                   mesh=_sc_mesh(), scratch_types=scratch)
    return fn(dst_f, jnp.zeros((NP, D), jnp.float32),
              jnp.ones((CH, D), jnp.float32))


def kernel(X, adj, y, enc0_Wl, enc0_bl, enc0_Wr, enc1_Wl, enc1_bl, enc1_Wr,
           enc2_Wl, enc2_bl, enc2_Wr, mt_Wl, mt_bl, mt_Wr, mt2_Wl, mt2_bl,
           mt2_Wr, mt3_Wl, mt3_bl, mt3_Wr):
    x = X[0, 0]
    x_pad = jnp.pad(x, ((0, NP - N), (0, 0)))
    src, dst = adj[0], adj[1]
    y_pad = jnp.pad(y[0], ((0, NP - N), (0, 0)))

    wl_cat = jnp.concatenate([mt_Wl, mt2_Wl, mt3_Wl], axis=1)
    wr_cat = jnp.concatenate([mt_Wr, mt2_Wr, mt3_Wr], axis=1)
    bl_cat = jnp.concatenate([mt_bl, mt2_bl, mt3_bl], axis=0)

    src_w = jnp.concatenate([src, jnp.zeros((EPAD - E,), jnp.int32)]
                            ).reshape(W, CPW, CH)
    dst_w = jnp.concatenate([dst, jnp.full((EPAD - E,), N, jnp.int32)]
                            ).reshape(W, CPW, CH)

    degf = _sc_deg_call(dst_w)
    acc0 = _sc_agg_call(x_pad, src_w, dst_w, D)
    h1, degc = _layer1_call(acc0, degf, x_pad, enc0_Wl, enc0_bl, enc0_Wr)
    deg2 = degc[None]
    acc1 = _sc_agg_call(h1, src_w, dst_w, D)
    h2 = _layer_call(acc1, deg2, h1, enc1_Wl, enc1_bl, enc1_Wr, "relu")
    acc2 = _sc_agg_call(h2, src_w, dst_w, D)
    h3 = _layer_call(acc2, deg2, h2, enc2_Wl, enc2_bl, enc2_Wr, "sigmoid")
    acch = _sc_agg_call(h3, src_w, dst_w, D)
    loss = _loss_call(acch, deg2, h3, wl_cat, wr_cat, bl_cat, y_pad)
    return loss[0, 0]


# speedup vs baseline: 1.4868x; 1.0009x over previous
import functools

import jax
import jax.numpy as jnp
from jax import lax
from jax.experimental import pallas as pl
from jax.experimental.pallas import tpu as pltpu

N = 10000
D = 128
NCL = 16
NH = 3 * NCL
NP = 10240
BR = 512
GRID = NP // BR

E = 320000
NC = 2
NS = 16
W = NC * NS
CH = 128
CPW = 79
EPAD = W * CPW * CH


def _sigmoid(x):
    return 1.0 / (1.0 + jnp.exp(-x))


def _layer_body(acc_ref, deg_ref, x_ref, wl_ref, bl_ref, wr_ref, o_ref, *, act):
    a = jnp.sum(acc_ref[...], axis=0)
    deg = jnp.sum(deg_ref[...], axis=0)[:, 0]
    mean = a / jnp.maximum(deg, 1.0)[:, None]
    out = (jnp.dot(mean, wl_ref[...], preferred_element_type=jnp.float32)
           + bl_ref[...]
           + jnp.dot(x_ref[...], wr_ref[...], preferred_element_type=jnp.float32))
    if act == "relu":
        out = jnp.maximum(out, 0.0)
    else:
        out = _sigmoid(out)
    o_ref[...] = out


def _layer1_body(acc_ref, degf_ref, x_ref, wl_ref, bl_ref, wr_ref,
                 o_ref, degc_ref):
    a = jnp.sum(acc_ref[...], axis=0)
    deg = degf_ref[0, :, 0] + degf_ref[1, :, 0]
    degc_ref[...] = jnp.broadcast_to(deg[:, None], (BR, NCL))
    mean = a / jnp.maximum(deg, 1.0)[:, None]
    out = (jnp.dot(mean, wl_ref[...], preferred_element_type=jnp.float32)
           + bl_ref[...]
           + jnp.dot(x_ref[...], wr_ref[...], preferred_element_type=jnp.float32))
    o_ref[...] = jnp.maximum(out, 0.0)


def _layer1_call(acc2, degf, x, wl, bl, wr):
    A = acc2.shape[0]
    return pl.pallas_call(
        _layer1_body,
        grid=(GRID,),
        in_specs=[
            pl.BlockSpec((A, BR, D), lambda i: (0, i, 0)),
            pl.BlockSpec((NC, BR, D), lambda i: (0, i, 0)),
            pl.BlockSpec((BR, D), lambda i: (i, 0)),
            pl.BlockSpec((D, D), lambda i: (0, 0)),
            pl.BlockSpec((1, D), lambda i: (0, 0)),
            pl.BlockSpec((D, D), lambda i: (0, 0)),
        ],
        out_specs=[
            pl.BlockSpec((BR, D), lambda i: (i, 0)),
            pl.BlockSpec((BR, NCL), lambda i: (i, 0)),
        ],
        out_shape=[
            jax.ShapeDtypeStruct((NP, D), jnp.float32),
            jax.ShapeDtypeStruct((NP, NCL), jnp.float32),
        ],
    )(acc2, degf, x, wl, bl.reshape(1, -1), wr)


def _loss_body(acch_ref, deg_ref, h_ref, wlcat_ref, wrcat_ref, blcat_ref,
               y_ref, o_ref):
    i = pl.program_id(0)
    a = jnp.sum(acch_ref[...], axis=0)
    deg = jnp.sum(deg_ref[...], axis=0)[:, 0]
    mean_h = a / jnp.maximum(deg, 1.0)[:, None]
    o = _sigmoid(jnp.dot(mean_h, wlcat_ref[...], preferred_element_type=jnp.float32)
                 + blcat_ref[...]
                 + jnp.dot(h_ref[...], wrcat_ref[...], preferred_element_type=jnp.float32))
    rows = i * BR + lax.broadcasted_iota(jnp.int32, (BR, 1), 0)
    valid = rows < N
    cls = lax.broadcasted_iota(jnp.int32, (BR, NCL), 1)
    total = jnp.float32(0.0)
    for k in range(3):
        lg = o[:, k * NCL:(k + 1) * NCL]
        m = jnp.max(lg, axis=1, keepdims=True)
        lse = jnp.log(jnp.sum(jnp.exp(lg - m), axis=1, keepdims=True)) + m
        logp = lg - lse
        sel = jnp.sum(jnp.where(cls == y_ref[:, k:k + 1], logp, 0.0),
                      axis=1, keepdims=True)
        total = total + jnp.sum(jnp.where(valid, sel, 0.0))

    @pl.when(i == 0)
    def _():
        o_ref[...] = jnp.zeros_like(o_ref)

    o_ref[...] += jnp.reshape(-total / N, (1, 1))


def _layer_call(acc2, deg2, x, wl, bl, wr, act):
    A = acc2.shape[0]
    dout = wl.shape[1]
    return pl.pallas_call(
        functools.partial(_layer_body, act=act),
        grid=(GRID,),
        in_specs=[
            pl.BlockSpec((A, BR, D), lambda i: (0, i, 0)),
            pl.BlockSpec((A, BR, NCL), lambda i: (0, i, 0)),
            pl.BlockSpec((BR, D), lambda i: (i, 0)),
            pl.BlockSpec((D, dout), lambda i: (0, 0)),
            pl.BlockSpec((1, dout), lambda i: (0, 0)),
            pl.BlockSpec((D, dout), lambda i: (0, 0)),
        ],
        out_specs=pl.BlockSpec((BR, dout), lambda i: (i, 0)),
        out_shape=jax.ShapeDtypeStruct((NP, dout), jnp.float32),
    )(acc2, deg2, x, wl, bl.reshape(1, -1), wr)


def _loss_call(acch2, deg2, h, wlcat, wrcat, blcat, y):
    A = acch2.shape[0]
    return pl.pallas_call(
        _loss_body,
        grid=(GRID,),
        in_specs=[
            pl.BlockSpec((A, BR, D), lambda i: (0, i, 0)),
            pl.BlockSpec((A, BR, NCL), lambda i: (0, i, 0)),
            pl.BlockSpec((BR, D), lambda i: (i, 0)),
            pl.BlockSpec((D, NH), lambda i: (0, 0)),
            pl.BlockSpec((D, NH), lambda i: (0, 0)),
            pl.BlockSpec((1, NH), lambda i: (0, 0)),
            pl.BlockSpec((BR, 3), lambda i: (i, 0)),
        ],
        out_specs=pl.BlockSpec((1, 1), lambda i: (0, 0)),
        out_shape=jax.ShapeDtypeStruct((1, 1), jnp.float32),
    )(acch2, deg2, h, wlcat, wrcat, blcat.reshape(1, -1), y)


from jax.experimental.pallas import tpu_sc as plsc

RT = NP // NS


def _sc_mesh():
    return plsc.VectorSubcoreMesh(core_axis_name="c", subcore_axis_name="s",
                                  num_cores=NC, num_subcores=NS)


NCH = EPAD // CH


def _worker_start(c, s):
    wid = s * NC + c
    return wid * CPW


def _sc_agg_call(x_pad, src_f, dst_f, d):
    scratch = [
        pltpu.VMEM((CPW, CH), jnp.int32),
        pltpu.VMEM((CPW, CH), jnp.int32),
        pltpu.VMEM((CH, d), jnp.float32),
        pltpu.VMEM_SHARED((NP, d), jnp.float32),
        pltpu.SemaphoreType.DMA,
    ]

    def body(x_hbm, src_hbm, dst_hbm, zrows_hbm, out_hbm,
             src_v, dst_v, rows_v, acc, sem):
        c = lax.axis_index("c")
        s = lax.axis_index("s")
        base = s * RT
        wid = s * NC + c
        pltpu.sync_copy(zrows_hbm.at[pl.ds(base, RT)], acc.at[pl.ds(base, RT)])
        pltpu.sync_copy(src_hbm.at[wid], src_v)
        pltpu.sync_copy(dst_hbm.at[wid], dst_v)
        plsc.subcore_barrier()

        @pl.loop(0, CPW)
        def _chunk(j):
            pltpu.async_copy(x_hbm.at[src_v.at[j]], rows_v, sem).wait()
            pltpu.sync_copy(rows_v, acc.at[dst_v.at[j]], add=True)

        plsc.subcore_barrier()
        pltpu.sync_copy(acc.at[pl.ds(base, RT)], out_hbm.at[c, pl.ds(base, RT)])

    fn = pl.kernel(body,
                   out_type=jax.ShapeDtypeStruct((NC, NP, d), jnp.float32),
                   mesh=_sc_mesh(), scratch_types=scratch)
    return fn(x_pad, src_f, dst_f, jnp.zeros((NP, d), jnp.float32))


def _sc_deg_call(dst_f):
    scratch = [
        pltpu.VMEM((CPW, CH), jnp.int32),
        pltpu.VMEM((CH, D), jnp.float32),
        pltpu.VMEM_SHARED((NP, D), jnp.float32),
    ]

    def body(dst_hbm, zdeg_hbm, ones_hbm, degout_hbm, dst_v, ones_v, dacc):
        c = lax.axis_index("c")
        s = lax.axis_index("s")
        base = s * RT
        wid = s * NC + c
        pltpu.sync_copy(zdeg_hbm.at[pl.ds(base, RT)], dacc.at[pl.ds(base, RT)])
        pltpu.sync_copy(ones_hbm, ones_v)
        pltpu.sync_copy(dst_hbm.at[wid], dst_v)
        plsc.subcore_barrier()

        @pl.loop(0, CPW)
        def _chunk(j):
            pltpu.sync_copy(ones_v, dacc.at[dst_v.at[j]], add=True)

        plsc.subcore_barrier()
        pltpu.sync_copy(dacc.at[pl.ds(base, RT)],
                        degout_hbm.at[c, pl.ds(base, RT)])

    fn = pl.kernel(body,
                   out_type=jax.ShapeDtypeStruct((NC, NP, D), jnp.float32),
                   mesh=_sc_mesh(), scratch_types=scratch)
    return fn(dst_f, jnp.zeros((NP, D), jnp.float32),
              jnp.ones((CH, D), jnp.float32))


def kernel(X, adj, y, enc0_Wl, enc0_bl, enc0_Wr, enc1_Wl, enc1_bl, enc1_Wr,
           enc2_Wl, enc2_bl, enc2_Wr, mt_Wl, mt_bl, mt_Wr, mt2_Wl, mt2_bl,
           mt2_Wr, mt3_Wl, mt3_bl, mt3_Wr):
    x = X[0, 0]
    x_pad = jnp.pad(x, ((0, NP - N), (0, 0)))
    src, dst = adj[0], adj[1]
    y_pad = jnp.pad(y[0], ((0, NP - N), (0, 0)))

    wl_cat = jnp.concatenate([mt_Wl, mt2_Wl, mt3_Wl], axis=1)
    wr_cat = jnp.concatenate([mt_Wr, mt2_Wr, mt3_Wr], axis=1)
    bl_cat = jnp.concatenate([mt_bl, mt2_bl, mt3_bl], axis=0)

    pad_dst = N + (jnp.arange(EPAD - E, dtype=jnp.int32) % (NP - N))
    src_w = jnp.concatenate([src, jnp.zeros((EPAD - E,), jnp.int32)]
                            ).reshape(W, CPW, CH)
    dst_w = jnp.concatenate([dst, pad_dst]).reshape(W, CPW, CH)

    degf = _sc_deg_call(dst_w)
    acc0 = _sc_agg_call(x_pad, src_w, dst_w, D)
    h1, degc = _layer1_call(acc0, degf, x_pad, enc0_Wl, enc0_bl, enc0_Wr)
    deg2 = degc[None]
    acc1 = _sc_agg_call(h1, src_w, dst_w, D)
    h2 = _layer_call(acc1, deg2, h1, enc1_Wl, enc1_bl, enc1_Wr, "relu")
    acc2 = _sc_agg_call(h2, src_w, dst_w, D)
    h3 = _layer_call(acc2, deg2, h2, enc2_Wl, enc2_bl, enc2_Wr, "sigmoid")
    acch = _sc_agg_call(h3, src_w, dst_w, D)
    loss = _loss_call(acch, deg2, h3, wl_cat, wr_cat, bl_cat, y_pad)
    return loss[0, 0]
